# Initial kernel scaffold; baseline (speedup 1.0000x reference)
import functools

import jax
import jax.numpy as jnp
from jax import lax
from jax.experimental import pallas as pl
from jax.experimental.pallas import tpu as pltpu
from jax.experimental.pallas import tpu_sc as plsc

_N = 10000
_E = 320000
_D = 128

_NC = 2
_NS = 16
_NW = _NC * _NS
_K = 128
_G = 79
_EPAD = _NW * _G * _K
_NP = 10016
_ROWS_PER_SUB = _NP // _NS
_PAD_DST = _NP - 1


def _sc_body(h_hbm, src_hbm, dst_hbm, psum_hbm, pdeg_hbm,
             src_v, dst_v, rows_v, zeros16_v, ones16_v, sem):
    c = lax.axis_index("c")
    s = lax.axis_index("s")
    wid = c * _NS + s

    def _scoped(acc_sh, deg_sh):
        zvec = jnp.zeros((16,), jnp.float32)
        ovec = jnp.ones((16,), jnp.float32)

        def _zero_bufs(i, carry):
            for j in range(_D // 16):
                rows_v[i, pl.ds(j * 16, 16)] = zvec
            zeros16_v[i, pl.ds(0, 16)] = zvec
            ones16_v[i, pl.ds(0, 16)] = ovec
            return carry

        lax.fori_loop(0, _K, _zero_bufs, 0)

        base = s * _ROWS_PER_SUB
        for k in range(_ROWS_PER_SUB // _K):
            pltpu.sync_copy(rows_v, acc_sh.at[pl.ds(base + k * _K, _K)])
            pltpu.sync_copy(zeros16_v, deg_sh.at[pl.ds(base + k * _K, _K)])
        rem = _ROWS_PER_SUB % _K
        if rem:
            off = base + (_ROWS_PER_SUB // _K) * _K
            pltpu.sync_copy(rows_v.at[pl.ds(0, rem)], acc_sh.at[pl.ds(off, rem)])
            pltpu.sync_copy(zeros16_v.at[pl.ds(0, rem)], deg_sh.at[pl.ds(off, rem)])

        plsc.subcore_barrier()

        pltpu.sync_copy(src_hbm.at[wid], src_v)
        pltpu.sync_copy(dst_hbm.at[wid], dst_v)

        def _edge_chunk(g, carry):
            pltpu.async_copy(h_hbm.at[src_v.at[g]], rows_v, sem).wait()
            pltpu.sync_copy(rows_v, acc_sh.at[dst_v.at[g]], add=True)
            pltpu.sync_copy(ones16_v, deg_sh.at[dst_v.at[g]], add=True)
            return carry

        lax.fori_loop(0, _G, _edge_chunk, 0)

        plsc.subcore_barrier()

        pltpu.sync_copy(acc_sh.at[pl.ds(base, _ROWS_PER_SUB)], psum_hbm.at[c, s])
        pltpu.sync_copy(deg_sh.at[pl.ds(base, _ROWS_PER_SUB)], pdeg_hbm.at[c, s])

    pl.run_scoped(
        _scoped,
        pltpu.VMEM_SHARED((_NP, _D), jnp.float32),
        pltpu.VMEM_SHARED((_NP, 16), jnp.float32),
    )


@functools.partial(
    pl.kernel,
    out_type=(
        jax.ShapeDtypeStruct((_NC, _NS, _ROWS_PER_SUB, _D), jnp.float32),
        jax.ShapeDtypeStruct((_NC, _NS, _ROWS_PER_SUB, 16), jnp.float32),
    ),
    mesh=plsc.VectorSubcoreMesh(core_axis_name="c", subcore_axis_name="s"),
    scratch_types=[
        pltpu.VMEM((_G, _K), jnp.int32),
        pltpu.VMEM((_G, _K), jnp.int32),
        pltpu.VMEM((_K, _D), jnp.float32),
        pltpu.VMEM((_K, 16), jnp.float32),
        pltpu.VMEM((_K, 16), jnp.float32),
        pltpu.SemaphoreType.DMA,
    ],
)
def _sc_scatter(h_hbm, src_hbm, dst_hbm, psum_hbm, pdeg_hbm,
                src_v, dst_v, rows_v, zeros16_v, ones16_v, sem):
    _sc_body(h_hbm, src_hbm, dst_hbm, psum_hbm, pdeg_hbm,
             src_v, dst_v, rows_v, zeros16_v, ones16_v, sem)


def _combine_body(h_ref, p0_ref, p1_ref, d0_ref, d1_ref, o_ref):
    deg = jnp.maximum(d0_ref[:, 0:1] + d1_ref[:, 0:1], 1.0)
    o_ref[...] = h_ref[...] - (p0_ref[...] + p1_ref[...]) / deg


def kernel(H, edge_index):
    src = edge_index[0].astype(jnp.int32)
    dst = edge_index[1].astype(jnp.int32)
    pad = _EPAD - _E
    src_p = jnp.concatenate([src, jnp.zeros((pad,), jnp.int32)])
    dst_p = jnp.concatenate([dst, jnp.full((pad,), _PAD_DST, jnp.int32)])
    src_p = src_p.reshape(_NW, _G, _K)
    dst_p = dst_p.reshape(_NW, _G, _K)

    psum, pdeg = _sc_scatter(H, src_p, dst_p)
    p = psum.reshape(_NC, _NP, _D)[:, :_N]
    d = pdeg.reshape(_NC, _NP, 16)[:, :_N]

    out = pl.pallas_call(
        _combine_body,
        out_shape=jax.ShapeDtypeStruct((_N, _D), jnp.float32),
    )(H, p[0], p[1], d[0], d[1])
    return out


# speedup vs baseline: 3.6876x; 3.6876x over previous
import functools

import jax
import jax.numpy as jnp
from jax import lax
from jax.experimental import pallas as pl
from jax.experimental.pallas import tpu as pltpu
from jax.experimental.pallas import tpu_sc as plsc

_N = 10000
_E = 320000
_D = 128

_NC = 2
_NS = 16
_NW = _NC * _NS
_K = 128
_G = 79
_EPAD = _NW * _G * _K
_NP = 10016
_RPS = _NP // _NS
_PAD_DST = _NP - 1


@functools.partial(
    pl.kernel,
    out_type=(
        jax.ShapeDtypeStruct((_NC, _NS, _RPS, _D), jnp.float32),
        jax.ShapeDtypeStruct((_NC, _NS, _RPS, _D), jnp.float32),
    ),
    mesh=plsc.VectorSubcoreMesh(core_axis_name="c", subcore_axis_name="s"),
    scratch_types=[
        pltpu.VMEM((_G, _K), jnp.int32),
        pltpu.VMEM((_G, _K), jnp.int32),
        pltpu.VMEM((_K, _D), jnp.float32),
        pltpu.VMEM_SHARED((_NP, _D), jnp.float32),
        pltpu.SemaphoreType.DMA,
    ],
)
def _sc_scatter(h_hbm, src_hbm, dst_hbm, psum_hbm, pdeg_hbm,
                src_v, dst_v, rows_v, acc_sh, sem):
    c = lax.axis_index("c")
    s = lax.axis_index("s")
    wid = c * _NS + s
    base = s * _RPS

    def _fill_rows(val):
        vec = jnp.full((16,), val, jnp.float32)

        def _body(i, carry):
            for j in range(_D // 16):
                rows_v[i, pl.ds(j * 16, 16)] = vec
            return carry

        lax.fori_loop(0, _K, _body, 0)

    def _zero_acc_slice():
        for k in range(_RPS // _K):
            pltpu.sync_copy(rows_v, acc_sh.at[pl.ds(base + k * _K, _K)])
        rem = _RPS % _K
        if rem:
            off = base + (_RPS // _K) * _K
            pltpu.sync_copy(rows_v.at[pl.ds(0, rem)], acc_sh.at[pl.ds(off, rem)])

    _fill_rows(0.0)
    _zero_acc_slice()

    pltpu.sync_copy(src_hbm.at[wid], src_v)
    pltpu.sync_copy(dst_hbm.at[wid], dst_v)

    plsc.subcore_barrier()

    def _edge_chunk(g, carry):
        pltpu.async_copy(h_hbm.at[src_v.at[g]], rows_v, sem).wait()
        pltpu.sync_copy(rows_v, acc_sh.at[dst_v.at[g]], add=True)
        return carry

    lax.fori_loop(0, _G, _edge_chunk, 0)

    plsc.subcore_barrier()

    pltpu.sync_copy(acc_sh.at[pl.ds(base, _RPS)], psum_hbm.at[c, s])

    plsc.subcore_barrier()

    _fill_rows(0.0)
    _zero_acc_slice()
    plsc.subcore_barrier()
    _fill_rows(1.0)

    def _deg_chunk(g, carry):
        pltpu.sync_copy(rows_v, acc_sh.at[dst_v.at[g]], add=True)
        return carry

    lax.fori_loop(0, _G, _deg_chunk, 0)

    plsc.subcore_barrier()

    pltpu.sync_copy(acc_sh.at[pl.ds(base, _RPS)], pdeg_hbm.at[c, s])


def _combine_body(h_ref, p0_ref, p1_ref, d0_ref, d1_ref, o_ref):
    deg = jnp.maximum(d0_ref[...] + d1_ref[...], 1.0)
    o_ref[...] = h_ref[...] - (p0_ref[...] + p1_ref[...]) / deg


def kernel(H, edge_index):
    src = edge_index[0].astype(jnp.int32)
    dst = edge_index[1].astype(jnp.int32)
    pad = _EPAD - _E
    src_p = jnp.concatenate([src, jnp.zeros((pad,), jnp.int32)])
    dst_p = jnp.concatenate([dst, jnp.full((pad,), _PAD_DST, jnp.int32)])
    src_p = src_p.reshape(_NW, _G, _K)
    dst_p = dst_p.reshape(_NW, _G, _K)

    psum, pdeg = _sc_scatter(H, src_p, dst_p)
    p = psum.reshape(_NC, _NP, _D)[:, :_N]
    d = pdeg.reshape(_NC, _NP, _D)[:, :_N, 0:1]

    out = pl.pallas_call(
        _combine_body,
        out_shape=jax.ShapeDtypeStruct((_N, _D), jnp.float32),
    )(H, p[0], p[1], d[0], d[1])
    return out
